# Initial kernel scaffold; baseline (speedup 1.0000x reference)
import jax
import jax.numpy as jnp
from jax.experimental import pallas as pl


def kernel(x, mask, W, b):
    raise NotImplementedError("write your pallas kernel here")


# speedup vs baseline: 3.7369x; 3.7369x over previous
import jax
import jax.numpy as jnp
from jax.experimental import pallas as pl

BH, BW = 16, 16
SH, SW = 14, 14
NBY, NBX = 28, 28
TOL = 0.999
H = W = 384
C = 192
OW = NBX * SW


def _stripe_kernel(ma_ref, mb_ref, xa_ref, xb_ref, w_ref, b_ref, o_ref):
    ma = ma_ref[0, 0]
    mb = mb_ref[0, 0][0:2]
    ind = (jnp.concatenate([ma, mb], axis=0) > TOL).astype(jnp.float32)
    s = jnp.sum(ind, axis=0, keepdims=True)
    c_io = jax.lax.broadcasted_iota(jnp.int32, (W, NBX), 0)
    j_io = jax.lax.broadcasted_iota(jnp.int32, (W, NBX), 1)
    memb = ((c_io >= j_io * SW) & (c_io < j_io * SW + BW)).astype(jnp.float32)
    cnt = jnp.dot(s, memb)
    gate28 = (cnt > 0.5).astype(jnp.float32)
    cc = jax.lax.broadcasted_iota(jnp.int32, (NBX, OW), 1)
    jj = jax.lax.broadcasted_iota(jnp.int32, (NBX, OW), 0)
    up = ((cc >= jj * SW) & (cc < jj * SW + SW)).astype(jnp.float32)
    g = jnp.dot(gate28, up)

    xa = xa_ref[0]
    xb = xb_ref[0][0:2]
    win = jnp.concatenate([xa, xb], axis=0)
    acc = jnp.zeros((SH, OW, C), jnp.float32)
    for dy in range(3):
        for dx in range(3):
            sl = win[dy:dy + SH, dx:dx + OW, :]
            acc = acc + jax.lax.dot_general(
                sl, w_ref[dy * 3 + dx],
                (((2,), (0,)), ((), ())),
                preferred_element_type=jnp.float32)
    acc = acc + b_ref[0][None, None, :]
    o_ref[0] = acc * g[0][None, :, None]


def kernel(x, mask, W_, b):
    B = x.shape[0]
    xp = jnp.pad(x.astype(jnp.bfloat16),
                 ((0, 0), (0, (NBY + 1) * SH - H), (0, 400 - W), (0, 0)))
    mp = jnp.pad(mask[..., 0], ((0, 0), (0, (NBY + 1) * SH - H), (0, 0)))
    mp = mp.reshape(B, NBY + 1, SH, W)
    wb = W_.astype(jnp.bfloat16).reshape(9, C, C)
    b2 = b.reshape(1, C)

    out = pl.pallas_call(
        _stripe_kernel,
        grid=(B, NBY),
        in_specs=[
            pl.BlockSpec((1, 1, SH, W), lambda bi, i: (bi, i, 0, 0)),
            pl.BlockSpec((1, 1, SH, W), lambda bi, i: (bi, i + 1, 0, 0)),
            pl.BlockSpec((1, SH, 400, C), lambda bi, i: (bi, i, 0, 0)),
            pl.BlockSpec((1, SH, 400, C), lambda bi, i: (bi, i + 1, 0, 0)),
            pl.BlockSpec((9, C, C), lambda bi, i: (0, 0, 0)),
            pl.BlockSpec((1, C), lambda bi, i: (0, 0)),
        ],
        out_specs=pl.BlockSpec((1, SH, OW, C), lambda bi, i: (bi, i, 0, 0)),
        out_shape=jax.ShapeDtypeStruct((B, H - 2, W - 2, C), jnp.float32),
    )(mp, mp, xp, xp, wb, b2)
    return out


# speedup vs baseline: 3.7370x; 1.0000x over previous
import jax
import jax.numpy as jnp
from jax.experimental import pallas as pl
from jax.experimental.pallas import tpu as pltpu

BH, BW = 16, 16
SH, SW = 14, 14
NBY, NBX = 28, 28
TOL = 0.999
H = W = 384
C = 192
OW = NBX * SW


def _stripe_kernel(ma_ref, mb_ref, xa_ref, xb_ref, w_ref, b_ref, o_ref):
    ma = ma_ref[0, 0]
    mb = mb_ref[0, 0][0:2]
    ind = (jnp.concatenate([ma, mb], axis=0) > TOL).astype(jnp.float32)
    s = jnp.sum(ind, axis=0, keepdims=True)
    c_io = jax.lax.broadcasted_iota(jnp.int32, (W, NBX), 0)
    j_io = jax.lax.broadcasted_iota(jnp.int32, (W, NBX), 1)
    memb = ((c_io >= j_io * SW) & (c_io < j_io * SW + BW)).astype(jnp.float32)
    cnt = jnp.dot(s, memb)
    gate28 = (cnt > 0.5).astype(jnp.float32)
    cc = jax.lax.broadcasted_iota(jnp.int32, (NBX, OW), 1)
    jj = jax.lax.broadcasted_iota(jnp.int32, (NBX, OW), 0)
    up = ((cc >= jj * SW) & (cc < jj * SW + SW)).astype(jnp.float32)
    g = jnp.dot(gate28, up)

    xa = xa_ref[0]
    xb = xb_ref[0][0:2]
    win = jnp.concatenate([xa, xb], axis=0)
    acc = jnp.zeros((SH, OW, C), jnp.float32)
    for dy in range(3):
        for dx in range(3):
            sl = win[dy:dy + SH, dx:dx + OW, :]
            acc = acc + jax.lax.dot_general(
                sl, w_ref[dy * 3 + dx],
                (((2,), (0,)), ((), ())),
                preferred_element_type=jnp.float32)
    acc = acc + b_ref[0][None, None, :]
    o_ref[0] = acc * g[0][None, :, None]


def kernel(x, mask, W_, b):
    B = x.shape[0]
    xp = jnp.pad(x.astype(jnp.bfloat16),
                 ((0, 0), (0, (NBY + 1) * SH - H), (0, 400 - W), (0, 0)))
    mp = jnp.pad(mask[..., 0], ((0, 0), (0, (NBY + 1) * SH - H), (0, 0)))
    mp = mp.reshape(B, NBY + 1, SH, W)
    wb = W_.astype(jnp.bfloat16).reshape(9, C, C)
    b2 = b.reshape(1, C)

    out = pl.pallas_call(
        _stripe_kernel,
        grid=(B, NBY),
        in_specs=[
            pl.BlockSpec((1, 1, SH, W), lambda bi, i: (bi, i, 0, 0)),
            pl.BlockSpec((1, 1, SH, W), lambda bi, i: (bi, i + 1, 0, 0)),
            pl.BlockSpec((1, SH, 400, C), lambda bi, i: (bi, i, 0, 0)),
            pl.BlockSpec((1, SH, 400, C), lambda bi, i: (bi, i + 1, 0, 0)),
            pl.BlockSpec((9, C, C), lambda bi, i: (0, 0, 0)),
            pl.BlockSpec((1, C), lambda bi, i: (0, 0)),
        ],
        out_specs=pl.BlockSpec((1, SH, OW, C), lambda bi, i: (bi, i, 0, 0)),
        out_shape=jax.ShapeDtypeStruct((B, H - 2, W - 2, C), jnp.float32),
        compiler_params=pltpu.CompilerParams(
            dimension_semantics=("parallel", "parallel")),
    )(mp, mp, xp, xp, wb, b2)
    return out


# speedup vs baseline: 4.4727x; 1.1969x over previous
import jax
import jax.numpy as jnp
from jax.experimental import pallas as pl
from jax.experimental.pallas import tpu as pltpu

TOL = 0.999
H = W = 384
C = 192
NB = 28
NBLK = 2 * NB * NB
NP = NB // 2
G = 8
SLAB = 18
MROW = G * SLAB * 24
OW = NB * 14


def _flags_kernel(m_ref, f_ref):
    r_io = jax.lax.broadcasted_iota(jnp.int32, (NB, H), 1)
    i_io = jax.lax.broadcasted_iota(jnp.int32, (NB, H), 0)
    memb = ((r_io >= i_io * 14) & (r_io < i_io * 14 + 16)).astype(jnp.float32)
    c_io = jax.lax.broadcasted_iota(jnp.int32, (W, NB), 0)
    j_io = jax.lax.broadcasted_iota(jnp.int32, (W, NB), 1)
    membT = ((c_io >= j_io * 14) & (c_io < j_io * 14 + 16)).astype(jnp.float32)
    for b in range(2):
        ind = (m_ref[b] > TOL).astype(jnp.float32)
        t1 = jnp.dot(memb, ind, preferred_element_type=jnp.float32)
        cnt = jnp.dot(t1, membT, preferred_element_type=jnp.float32)
        f_ref[b] = (cnt > 0.5).astype(jnp.int32)


def _pair_kernel(flags, xa_ref, xb_ref, wc_ref, b_ref, o_ref, js, xgrp):
    bi = pl.program_id(0)
    i = pl.program_id(1)
    base = bi * (NB * NB) + i * (2 * NB)

    def pro(nn, c):
        f = flags[0, base + nn]
        @pl.when(f != 0)
        def _():
            js[c] = nn
        return c + f
    nj = jax.lax.fori_loop(0, 2 * NB, pro, 0)

    o_ref[...] = jnp.zeros((1, 2 * 14, OW, C), jnp.float32)
    ng = (nj + G - 1) // G

    def group(g, carry):
        slots = []
        for s in range(G):
            t = jnp.minimum(g * G + s, nj - 1)
            nn = js[t]
            p = nn // NB
            jv = nn - p * NB
            cj = jv * 14
            cg = pl.multiple_of(jnp.minimum((cj // 8) * 8, W - 24), 8)
            slots.append((p, cg, cj - cg))
            xgrp[pl.ds(s * SLAB, 14)] = (
                xa_ref[0, pl.ds(p * 14, 14), pl.ds(cg, 24), :]
                .astype(jnp.bfloat16))
            tail0 = xa_ref[0, 14:16, pl.ds(cg, 24), :]
            tail1 = xb_ref[0, 0, :, pl.ds(cg, 24), :]
            xgrp[pl.ds(s * SLAB + 14, 2)] = (
                jnp.where(p == 1, tail1, tail0).astype(jnp.bfloat16))
        xf = xgrp[...].reshape((G * SLAB + 3) * 24, C)
        lcat = MROW + 52
        xcat = jnp.concatenate(
            [xf[0:lcat], xf[1:lcat + 1], xf[2:lcat + 2]], axis=1)
        acc = b_ref[...].astype(jnp.float32)
        for dy in range(3):
            acc = acc + jax.lax.dot_general(
                xcat[dy * 24:dy * 24 + MROW], wc_ref[dy],
                (((1,), (0,)), ((), ())),
                preferred_element_type=jnp.float32)
        av = acc.reshape(G * SLAB, 24, C)
        io = jax.lax.broadcasted_iota(jnp.int32, (14, 24, C), 1)
        for s in range(G):
            p, cg, dv = slots[s]
            sub = av[s * SLAB:s * SLAB + 14, :, :]
            ro = pl.ds(p * 14, 14)
            co = pl.ds(cg, 24)
            prev = o_ref[0, ro, co, :]
            new = jnp.where((io >= dv) & (io < dv + 14), sub, prev)
            o_ref[0, ro, co, :] = new
        return carry

    jax.lax.fori_loop(0, ng, group, 0)


def kernel(x, mask, W_, b):
    B = x.shape[0]
    flags = pl.pallas_call(
        _flags_kernel,
        grid=(1,),
        in_specs=[pl.BlockSpec((B, H, W), lambda i: (0, 0, 0))],
        out_specs=pl.BlockSpec((B, NB, NB), lambda i: (0, 0, 0)),
        out_shape=jax.ShapeDtypeStruct((B, NB, NB), jnp.int32),
    )(mask[..., 0])
    flags = flags.reshape(1, NBLK)

    wc = W_.astype(jnp.bfloat16).reshape(3, 3 * C, C)
    b2 = b.reshape(1, C)
    xtop = jnp.stack(
        [x[:, 28:366:28], x[:, 29:367:28]], axis=2)

    out = pl.pallas_call(
        _pair_kernel,
        grid_spec=pltpu.PrefetchScalarGridSpec(
            num_scalar_prefetch=1,
            grid=(B, NP),
            in_specs=[
                pl.BlockSpec((1, 2 * 14, W, C),
                             lambda bi, i, *_: (bi, i, 0, 0)),
                pl.BlockSpec((1, 1, 2, W, C),
                             lambda bi, i, *_: (bi, jnp.minimum(i, 12),
                                                0, 0, 0)),
                pl.BlockSpec((3, 3 * C, C), lambda *_: (0, 0, 0)),
                pl.BlockSpec((1, C), lambda *_: (0, 0)),
            ],
            out_specs=pl.BlockSpec((1, 2 * 14, OW, C),
                                   lambda bi, i, *_: (bi, i, 0, 0)),
            scratch_shapes=[
                pltpu.SMEM((2 * NB,), jnp.int32),
                pltpu.VMEM((G * SLAB + 3, 24, C), jnp.bfloat16),
            ],
        ),
        out_shape=jax.ShapeDtypeStruct((B, H - 2, W - 2, C), jnp.float32),
    )(flags, x, xtop, wc, b2)
    return out
